# baseline (speedup 1.0000x reference)
import functools

import jax
import jax.numpy as jnp
from jax import lax
from jax.experimental import pallas as pl
from jax.experimental.pallas import tpu as pltpu
from jax.experimental.pallas import tpu_sc as plsc

VOCAB = 100000
DIM = 128
BATCH = 16384

NC, NS = 2, 16
NW = NC * NS
BPW = BATCH // NW
CHUNK = 32
NCHUNK = BPW // CHUNK
NSLOT = 2
NLANE = 16
NVEC = DIM // NLANE


def _body(w_in_hbm, w_out_hbm, idx_in_hbm, idx_out_hbm, out_hbm,
          idx_in_v, idx_out_v, rows, prods_v, out_v, sems, idx_sems,
          out_sem):
    wid = lax.axis_index("s") * NC + lax.axis_index("c")
    base = wid * BPW
    lane = lax.iota(jnp.int32, NLANE)
    zeros = jnp.zeros((NLANE,), jnp.int32)

    head = NSLOT * CHUNK
    di = pltpu.async_copy(idx_in_hbm.at[pl.ds(base, head)],
                          idx_in_v.at[pl.ds(0, head)], idx_sems[0])
    do = pltpu.async_copy(idx_out_hbm.at[pl.ds(base, head)],
                          idx_out_v.at[pl.ds(0, head)], idx_sems[1])
    di.wait()
    do.wait()

    rows_in_v, rows_out_v = rows

    def issue(c, s):
        pltpu.async_copy(
            w_in_hbm.at[idx_in_v.at[pl.ds(c * CHUNK, CHUNK)]],
            rows_in_v.at[s], sems.at[s])
        pltpu.async_copy(
            w_out_hbm.at[idx_out_v.at[pl.ds(c * CHUNK, CHUNK)]],
            rows_out_v.at[s], sems.at[s])

    issue(0, 0)
    issue(1, 1)

    di2 = pltpu.async_copy(idx_in_hbm.at[pl.ds(base + head, BPW - head)],
                           idx_in_v.at[pl.ds(head, BPW - head)], idx_sems[0])
    do2 = pltpu.async_copy(idx_out_hbm.at[pl.ds(base + head, BPW - head)],
                           idx_out_v.at[pl.ds(head, BPW - head)], idx_sems[1])
    di2.wait()
    do2.wait()

    def chunk_body(c, carry):
        s = c % NSLOT
        pltpu.make_async_copy(
            w_in_hbm.at[idx_in_v.at[pl.ds(0, CHUNK)]],
            rows_in_v.at[0], sems.at[s]).wait()
        pltpu.make_async_copy(
            w_out_hbm.at[idx_out_v.at[pl.ds(0, CHUNK)]],
            rows_out_v.at[0], sems.at[s]).wait()

        def group(g, carry2):
            base_p = g * NLANE

            @plsc.parallel_loop(0, NLANE, unroll=2)
            def _pair(p):
                pair = base_p + p
                acc = (rows_in_v[s, pair, pl.ds(0, NLANE)]
                       * rows_out_v[s, pair, pl.ds(0, NLANE)])
                for j in range(1, NVEC):
                    acc = acc + (rows_in_v[s, pair, pl.ds(j * NLANE, NLANE)]
                                 * rows_out_v[s, pair, pl.ds(j * NLANE, NLANE)])
                prods_v[p, :] = acc
            cols = [plsc.load_gather(
                prods_v, [lane, jnp.full((NLANE,), j, jnp.int32)])
                for j in range(NLANE)]
            while len(cols) > 1:
                cols = [a + b for a, b in zip(cols[::2], cols[1::2])]
            out_v[pl.ds(c * CHUNK + base_p, NLANE)] = cols[0]
            return carry2

        lax.fori_loop(0, CHUNK // NLANE, group, 0)

        @pl.when(c + NSLOT < NCHUNK)
        def _():
            issue(c + NSLOT, s)

        pltpu.async_copy(out_v.at[pl.ds(c * CHUNK, CHUNK)],
                         out_hbm.at[pl.ds(base + c * CHUNK, CHUNK)],
                         out_sem)

        return carry

    lax.fori_loop(0, NCHUNK, chunk_body, 0)

    pltpu.make_async_copy(out_v, out_hbm.at[pl.ds(base, BPW)],
                          out_sem).wait()


@functools.partial(
    pl.kernel,
    out_type=jax.ShapeDtypeStruct((BATCH,), jnp.float32),
    mesh=plsc.VectorSubcoreMesh(core_axis_name="c", subcore_axis_name="s"),
    compiler_params=pltpu.CompilerParams(needs_layout_passes=False),
    scratch_types=[
        pltpu.VMEM((BPW,), jnp.int32),
        pltpu.VMEM((BPW,), jnp.int32),
        pltpu.VMEM((NSLOT, CHUNK, DIM), jnp.float32),
        pltpu.VMEM((NSLOT, CHUNK, DIM), jnp.float32),
        pltpu.VMEM((NLANE, NLANE), jnp.float32),
        pltpu.VMEM((BPW,), jnp.float32),
        pltpu.SemaphoreType.DMA((NSLOT,)),
        pltpu.SemaphoreType.DMA,
        pltpu.SemaphoreType.DMA,
        pltpu.SemaphoreType.DMA,
    ],
)
def _sc_dot(w_in_hbm, w_out_hbm, idx_in_hbm, idx_out_hbm, out_hbm,
            idx_in_v, idx_out_v,
            rows_in3, rows_out3,
            prods_v, out_v, sem_ring, sem_2, sem_3, sem_4):
    _body(w_in_hbm, w_out_hbm, idx_in_hbm, idx_out_hbm, out_hbm,
          idx_in_v, idx_out_v,
          (rows_in3, rows_out3),
          prods_v, out_v, sem_ring, (sem_2, sem_3), sem_4)


def kernel(x, W_in, W_out):
    idx_in = x[:, 0].astype(jnp.int32)
    idx_out = x[:, 1].astype(jnp.int32)
    return _sc_dot(W_in, W_out, idx_in, idx_out)


# speedup vs baseline: 1.0646x; 1.0646x over previous
import functools

import jax
import jax.numpy as jnp
from jax import lax
from jax.experimental import pallas as pl
from jax.experimental.pallas import tpu as pltpu
from jax.experimental.pallas import tpu_sc as plsc

VOCAB = 100000
DIM = 128
BATCH = 16384

NC, NS = 2, 16
NW = NC * NS
BPW = BATCH // NW
CHUNK = 64
NCHUNK = BPW // CHUNK
NSLOT = 2
NLANE = 16
NVEC = DIM // NLANE


def _body(w_in_hbm, w_out_hbm, idx_in_hbm, idx_out_hbm, out_hbm,
          idx_in_v, idx_out_v, rows, prods_v, out_v, sems, idx_sems,
          out_sem):
    wid = lax.axis_index("s") * NC + lax.axis_index("c")
    base = wid * BPW
    lane = lax.iota(jnp.int32, NLANE)
    zeros = jnp.zeros((NLANE,), jnp.int32)

    head = NSLOT * CHUNK
    di = pltpu.async_copy(idx_in_hbm.at[pl.ds(base, head)],
                          idx_in_v.at[pl.ds(0, head)], idx_sems[0])
    do = pltpu.async_copy(idx_out_hbm.at[pl.ds(base, head)],
                          idx_out_v.at[pl.ds(0, head)], idx_sems[1])
    di.wait()
    do.wait()

    rows_in_v, rows_out_v = rows

    def issue(c, s):
        pltpu.async_copy(
            w_in_hbm.at[idx_in_v.at[pl.ds(c * CHUNK, CHUNK)]],
            rows_in_v.at[s], sems.at[s])
        pltpu.async_copy(
            w_out_hbm.at[idx_out_v.at[pl.ds(c * CHUNK, CHUNK)]],
            rows_out_v.at[s], sems.at[s])

    issue(0, 0)
    issue(1, 1)

    di2 = pltpu.async_copy(idx_in_hbm.at[pl.ds(base + head, BPW - head)],
                           idx_in_v.at[pl.ds(head, BPW - head)], idx_sems[0])
    do2 = pltpu.async_copy(idx_out_hbm.at[pl.ds(base + head, BPW - head)],
                           idx_out_v.at[pl.ds(head, BPW - head)], idx_sems[1])
    di2.wait()
    do2.wait()

    def chunk_body(c, carry):
        s = c % NSLOT
        pltpu.make_async_copy(
            w_in_hbm.at[idx_in_v.at[pl.ds(0, CHUNK)]],
            rows_in_v.at[0], sems.at[s]).wait()
        pltpu.make_async_copy(
            w_out_hbm.at[idx_out_v.at[pl.ds(0, CHUNK)]],
            rows_out_v.at[0], sems.at[s]).wait()

        def group(g, carry2):
            base_p = g * NLANE

            @plsc.parallel_loop(0, NLANE, unroll=2)
            def _pair(p):
                pair = base_p + p
                acc = (rows_in_v[s, pair, pl.ds(0, NLANE)]
                       * rows_out_v[s, pair, pl.ds(0, NLANE)])
                for j in range(1, NVEC):
                    acc = acc + (rows_in_v[s, pair, pl.ds(j * NLANE, NLANE)]
                                 * rows_out_v[s, pair, pl.ds(j * NLANE, NLANE)])
                prods_v[p, :] = acc
            cols = [plsc.load_gather(
                prods_v, [lane, jnp.full((NLANE,), j, jnp.int32)])
                for j in range(NLANE)]
            while len(cols) > 1:
                cols = [a + b for a, b in zip(cols[::2], cols[1::2])]
            out_v[pl.ds(c * CHUNK + base_p, NLANE)] = cols[0]
            return carry2

        lax.fori_loop(0, CHUNK // NLANE, group, 0)

        @pl.when(c + NSLOT < NCHUNK)
        def _():
            issue(c + NSLOT, s)

        pltpu.async_copy(out_v.at[pl.ds(c * CHUNK, CHUNK)],
                         out_hbm.at[pl.ds(base + c * CHUNK, CHUNK)],
                         out_sem)

        return carry

    lax.fori_loop(0, NCHUNK, chunk_body, 0)

    pltpu.make_async_copy(out_v, out_hbm.at[pl.ds(base, BPW)],
                          out_sem).wait()


@functools.partial(
    pl.kernel,
    out_type=jax.ShapeDtypeStruct((BATCH,), jnp.float32),
    mesh=plsc.VectorSubcoreMesh(core_axis_name="c", subcore_axis_name="s"),
    compiler_params=pltpu.CompilerParams(needs_layout_passes=False),
    scratch_types=[
        pltpu.VMEM((BPW,), jnp.int32),
        pltpu.VMEM((BPW,), jnp.int32),
        pltpu.VMEM((NSLOT, CHUNK, DIM), jnp.float32),
        pltpu.VMEM((NSLOT, CHUNK, DIM), jnp.float32),
        pltpu.VMEM((NLANE, NLANE), jnp.float32),
        pltpu.VMEM((BPW,), jnp.float32),
        pltpu.SemaphoreType.DMA((NSLOT,)),
        pltpu.SemaphoreType.DMA,
        pltpu.SemaphoreType.DMA,
        pltpu.SemaphoreType.DMA,
    ],
)
def _sc_dot(w_in_hbm, w_out_hbm, idx_in_hbm, idx_out_hbm, out_hbm,
            idx_in_v, idx_out_v,
            rows_in3, rows_out3,
            prods_v, out_v, sem_ring, sem_2, sem_3, sem_4):
    _body(w_in_hbm, w_out_hbm, idx_in_hbm, idx_out_hbm, out_hbm,
          idx_in_v, idx_out_v,
          (rows_in3, rows_out3),
          prods_v, out_v, sem_ring, (sem_2, sem_3), sem_4)


def kernel(x, W_in, W_out):
    idx_in = x[:, 0].astype(jnp.int32)
    idx_out = x[:, 1].astype(jnp.int32)
    return _sc_dot(W_in, W_out, idx_in, idx_out)
